# Initial kernel scaffold; baseline (speedup 1.0000x reference)
import jax
import jax.numpy as jnp
from jax.experimental import pallas as pl


def kernel(sampleU, sampleI, sampleR, betaU, betaI, alpha):
    raise NotImplementedError("write your pallas kernel here")


# speedup vs baseline: 1.1969x; 1.1969x over previous
import functools

import jax
import jax.numpy as jnp
from jax import lax
from jax.experimental import pallas as pl
from jax.experimental.pallas import tpu as pltpu
from jax.experimental.pallas import tpu_sc as plsc

B = 16384
NC = 2
NS = 16
NW = NC * NS
LANES = 16
ROW = 128
NROWS = B // ROW
ROWS_PER_W = NROWS // NW


def _sc_body(uidx_hbm, iidx_hbm, r_hbm, bu_hbm, bi_hbm, out_hbm,
             idxu_v, idxi_v, r_v, gu_v, gi_v, outv, sem):
    cid = lax.axis_index("c")
    sid = lax.axis_index("s")
    wid = sid * NC + cid
    rowbase = wid * ROWS_PER_W

    pltpu.sync_copy(uidx_hbm.at[pl.ds(rowbase, ROWS_PER_W)], idxu_v)
    pltpu.sync_copy(iidx_hbm.at[pl.ds(rowbase, ROWS_PER_W)], idxi_v)
    pltpu.sync_copy(r_hbm.at[pl.ds(rowbase, ROWS_PER_W)], r_v)

    copies = []
    for j in range(ROWS_PER_W):
        copies.append(pltpu.async_copy(bu_hbm.at[idxu_v.at[j]],
                                       gu_v.at[j], sem))
        copies.append(pltpu.async_copy(bi_hbm.at[idxi_v.at[j]],
                                       gi_v.at[j], sem))
    for c in copies:
        c.wait()

    acc2 = jnp.zeros((LANES,), jnp.float32)
    acc1 = jnp.zeros((LANES,), jnp.float32)
    for j in range(ROWS_PER_W):
        for k in range(ROW // LANES):
            sl = pl.ds(k * LANES, LANES)
            d = gu_v[j, sl] + gi_v[j, sl] - r_v[j, sl]
            acc2 = acc2 + d * d
            acc1 = acc1 + d

    outv[...] = acc2
    pltpu.sync_copy(outv, out_hbm.at[wid])
    outv[...] = acc1
    pltpu.sync_copy(outv, out_hbm.at[NW + wid])


@jax.jit
def _lfm_loss(sampleU, sampleI, sampleR, betaU, betaI, alpha):
    u2 = sampleU.reshape(NROWS, ROW)
    i2 = sampleI.reshape(NROWS, ROW)
    r2 = sampleR.reshape(NROWS, ROW)

    k = functools.partial(
        pl.kernel,
        out_type=jax.ShapeDtypeStruct((2 * NW, LANES), jnp.float32),
        mesh=plsc.VectorSubcoreMesh(core_axis_name="c",
                                    subcore_axis_name="s",
                                    num_cores=NC),
        scratch_types=[
            pltpu.VMEM((ROWS_PER_W, ROW), jnp.int32),
            pltpu.VMEM((ROWS_PER_W, ROW), jnp.int32),
            pltpu.VMEM((ROWS_PER_W, ROW), jnp.float32),
            pltpu.VMEM((ROWS_PER_W, ROW), jnp.float32),
            pltpu.VMEM((ROWS_PER_W, ROW), jnp.float32),
            pltpu.VMEM((LANES,), jnp.float32),
            pltpu.SemaphoreType.DMA,
        ],
    )(_sc_body)
    partials = k(u2, i2, r2, betaU, betaI)

    s2 = jnp.sum(partials[:NW])
    s1 = jnp.sum(partials[NW:])
    n = jnp.float32(B)
    return 0.5 * (s2 + 2.0 * alpha * s1 + n * alpha * alpha) / n


def kernel(sampleU, sampleI, sampleR, betaU, betaI, alpha):
    return _lfm_loss(sampleU, sampleI, sampleR, betaU, betaI, alpha)


# speedup vs baseline: 1.2421x; 1.0378x over previous
import functools

import jax
import jax.numpy as jnp
from jax import lax
from jax.experimental import pallas as pl
from jax.experimental.pallas import tpu as pltpu
from jax.experimental.pallas import tpu_sc as plsc

B = 16384
NC = 2
NS = 16
NW = NC * NS
LANES = 16
ROW = 128
NROWS = B // ROW
ROWS_PER_W = NROWS // NW


def _sc_body(uidx_hbm, iidx_hbm, r_hbm, bu_hbm, bi_hbm, out_hbm,
             idxu_v, idxi_v, r_v, gu_v, gi_v, outv, sem, gsem):
    cid = lax.axis_index("c")
    sid = lax.axis_index("s")
    wid = sid * NC + cid
    rowbase = wid * ROWS_PER_W
    rows = pl.ds(rowbase, ROWS_PER_W)

    cu = pltpu.async_copy(uidx_hbm.at[rows], idxu_v, sem)
    ci = pltpu.async_copy(iidx_hbm.at[rows], idxi_v, sem)
    cr = pltpu.async_copy(r_hbm.at[rows], r_v, sem)
    cu.wait()
    ci.wait()

    copies = []
    for j in range(ROWS_PER_W):
        copies.append(pltpu.async_copy(bu_hbm.at[idxu_v.at[j]],
                                       gu_v.at[j], gsem))
        copies.append(pltpu.async_copy(bi_hbm.at[idxi_v.at[j]],
                                       gi_v.at[j], gsem))
    cr.wait()
    for c in copies:
        c.wait()

    acc2 = jnp.zeros((LANES,), jnp.float32)
    acc1 = jnp.zeros((LANES,), jnp.float32)
    for j in range(ROWS_PER_W):
        for k in range(ROW // LANES):
            sl = pl.ds(k * LANES, LANES)
            d = gu_v[j, sl] + gi_v[j, sl] - r_v[j, sl]
            acc2 = acc2 + d * d
            acc1 = acc1 + d

    outv[0, :] = acc2
    outv[1, :] = acc1
    pltpu.sync_copy(outv, out_hbm.at[wid])


@jax.jit
def _lfm_loss(sampleU, sampleI, sampleR, betaU, betaI, alpha):
    u2 = sampleU.reshape(NROWS, ROW)
    i2 = sampleI.reshape(NROWS, ROW)
    r2 = sampleR.reshape(NROWS, ROW)

    k = functools.partial(
        pl.kernel,
        out_type=jax.ShapeDtypeStruct((NW, 2, LANES), jnp.float32),
        mesh=plsc.VectorSubcoreMesh(core_axis_name="c",
                                    subcore_axis_name="s",
                                    num_cores=NC),
        scratch_types=[
            pltpu.VMEM((ROWS_PER_W, ROW), jnp.int32),
            pltpu.VMEM((ROWS_PER_W, ROW), jnp.int32),
            pltpu.VMEM((ROWS_PER_W, ROW), jnp.float32),
            pltpu.VMEM((ROWS_PER_W, ROW), jnp.float32),
            pltpu.VMEM((ROWS_PER_W, ROW), jnp.float32),
            pltpu.VMEM((2, LANES), jnp.float32),
            pltpu.SemaphoreType.DMA,
            pltpu.SemaphoreType.DMA,
        ],
    )(_sc_body)
    partials = k(u2, i2, r2, betaU, betaI)

    s2 = jnp.sum(partials[:, 0, :])
    s1 = jnp.sum(partials[:, 1, :])
    n = jnp.float32(B)
    return 0.5 * (s2 + 2.0 * alpha * s1 + n * alpha * alpha) / n


def kernel(sampleU, sampleI, sampleR, betaU, betaI, alpha):
    return _lfm_loss(sampleU, sampleI, sampleR, betaU, betaI, alpha)


# speedup vs baseline: 1.2421x; 1.0000x over previous
import functools

import jax
import jax.numpy as jnp
from jax import lax
from jax.experimental import pallas as pl
from jax.experimental.pallas import tpu as pltpu
from jax.experimental.pallas import tpu_sc as plsc

B = 16384
NC = 2
NS = 16
NW = NC * NS
LANES = 16
PER_W = B // NW


def _sc_body(uidx_hbm, iidx_hbm, r_hbm, bu_hbm, bi_hbm, out_hbm,
             idxu_v, idxi_v, r_v, gu_v, gi_v, outv, sem, gsem):
    cid = lax.axis_index("c")
    sid = lax.axis_index("s")
    wid = sid * NC + cid
    sl_in = pl.ds(wid * PER_W, PER_W)

    cu = pltpu.async_copy(uidx_hbm.at[sl_in], idxu_v, sem)
    ci = pltpu.async_copy(iidx_hbm.at[sl_in], idxi_v, sem)
    cr = pltpu.async_copy(r_hbm.at[sl_in], r_v, sem)
    cu.wait()
    ci.wait()

    gu = pltpu.async_copy(bu_hbm.at[idxu_v], gu_v, gsem)
    gi = pltpu.async_copy(bi_hbm.at[idxi_v], gi_v, gsem)
    cr.wait()
    gu.wait()
    gi.wait()

    acc2 = jnp.zeros((LANES,), jnp.float32)
    acc1 = jnp.zeros((LANES,), jnp.float32)
    for k in range(PER_W // LANES):
        sl = pl.ds(k * LANES, LANES)
        d = gu_v[sl] + gi_v[sl] - r_v[sl]
        acc2 = acc2 + d * d
        acc1 = acc1 + d

    outv[0, :] = acc2
    outv[1, :] = acc1
    pltpu.sync_copy(outv, out_hbm.at[wid])


@jax.jit
def _lfm_loss(sampleU, sampleI, sampleR, betaU, betaI, alpha):
    k = functools.partial(
        pl.kernel,
        out_type=jax.ShapeDtypeStruct((NW, 2, LANES), jnp.float32),
        mesh=plsc.VectorSubcoreMesh(core_axis_name="c",
                                    subcore_axis_name="s",
                                    num_cores=NC),
        scratch_types=[
            pltpu.VMEM((PER_W,), jnp.int32),
            pltpu.VMEM((PER_W,), jnp.int32),
            pltpu.VMEM((PER_W,), jnp.float32),
            pltpu.VMEM((PER_W,), jnp.float32),
            pltpu.VMEM((PER_W,), jnp.float32),
            pltpu.VMEM((2, LANES), jnp.float32),
            pltpu.SemaphoreType.DMA,
            pltpu.SemaphoreType.DMA,
        ],
    )(_sc_body)
    partials = k(sampleU, sampleI, sampleR, betaU, betaI)

    s2 = jnp.sum(partials[:, 0, :])
    s1 = jnp.sum(partials[:, 1, :])
    n = jnp.float32(B)
    return 0.5 * (s2 + 2.0 * alpha * s1 + n * alpha * alpha) / n


def kernel(sampleU, sampleI, sampleR, betaU, betaI, alpha):
    return _lfm_loss(sampleU, sampleI, sampleR, betaU, betaI, alpha)


# speedup vs baseline: 1.2468x; 1.0038x over previous
import functools

import jax
import jax.numpy as jnp
from jax import lax
from jax.experimental import pallas as pl
from jax.experimental.pallas import tpu as pltpu
from jax.experimental.pallas import tpu_sc as plsc

B = 16384
NC = 2
NS = 16
NW = NC * NS
LANES = 16
PER_W = B // NW


def _sc_body(uidx_hbm, iidx_hbm, r_hbm, bu_hbm, bi_hbm, out_hbm,
             idxu_v, idxi_v, r_v, gu_v, gi_v, outv, sem, gsem):
    cid = lax.axis_index("c")
    sid = lax.axis_index("s")
    wid = sid * NC + cid
    sl_in = pl.ds(wid * PER_W, PER_W)

    cu = pltpu.async_copy(uidx_hbm.at[sl_in], idxu_v, sem)
    ci = pltpu.async_copy(iidx_hbm.at[sl_in], idxi_v, sem)
    cr = pltpu.async_copy(r_hbm.at[sl_in], r_v, sem)
    cu.wait()
    ci.wait()

    HALF = PER_W // 2
    h0, h1 = pl.ds(0, HALF), pl.ds(HALF, HALF)
    gu0 = pltpu.async_copy(bu_hbm.at[idxu_v.at[h0]], gu_v.at[h0], gsem)
    gi0 = pltpu.async_copy(bi_hbm.at[idxi_v.at[h0]], gi_v.at[h0], gsem)
    gu1 = pltpu.async_copy(bu_hbm.at[idxu_v.at[h1]], gu_v.at[h1], gsem)
    gi1 = pltpu.async_copy(bi_hbm.at[idxi_v.at[h1]], gi_v.at[h1], gsem)
    cr.wait()
    gu0.wait()
    gi0.wait()

    acc2 = jnp.zeros((LANES,), jnp.float32)
    acc1 = jnp.zeros((LANES,), jnp.float32)
    for k in range(HALF // LANES):
        sl = pl.ds(k * LANES, LANES)
        d = gu_v[sl] + gi_v[sl] - r_v[sl]
        acc2 = acc2 + d * d
        acc1 = acc1 + d

    gu1.wait()
    gi1.wait()
    for k in range(HALF // LANES, PER_W // LANES):
        sl = pl.ds(k * LANES, LANES)
        d = gu_v[sl] + gi_v[sl] - r_v[sl]
        acc2 = acc2 + d * d
        acc1 = acc1 + d

    outv[0, :] = acc2
    outv[1, :] = acc1
    pltpu.sync_copy(outv, out_hbm.at[wid])


@jax.jit
def _lfm_loss(sampleU, sampleI, sampleR, betaU, betaI, alpha):
    k = functools.partial(
        pl.kernel,
        out_type=jax.ShapeDtypeStruct((NW, 2, LANES), jnp.float32),
        mesh=plsc.VectorSubcoreMesh(core_axis_name="c",
                                    subcore_axis_name="s",
                                    num_cores=NC),
        scratch_types=[
            pltpu.VMEM((PER_W,), jnp.int32),
            pltpu.VMEM((PER_W,), jnp.int32),
            pltpu.VMEM((PER_W,), jnp.float32),
            pltpu.VMEM((PER_W,), jnp.float32),
            pltpu.VMEM((PER_W,), jnp.float32),
            pltpu.VMEM((2, LANES), jnp.float32),
            pltpu.SemaphoreType.DMA,
            pltpu.SemaphoreType.DMA,
        ],
    )(_sc_body)
    partials = k(sampleU, sampleI, sampleR, betaU, betaI)

    s2 = jnp.sum(partials[:, 0, :])
    s1 = jnp.sum(partials[:, 1, :])
    n = jnp.float32(B)
    return 0.5 * (s2 + 2.0 * alpha * s1 + n * alpha * alpha) / n


def kernel(sampleU, sampleI, sampleR, betaU, betaI, alpha):
    return _lfm_loss(sampleU, sampleI, sampleR, betaU, betaI, alpha)
